# baseline (speedup 1.0000x reference)
import functools

import jax
import jax.numpy as jnp
from jax import lax
from jax.experimental import pallas as pl
from jax.experimental.pallas import tpu as pltpu
from jax.experimental.pallas import tpu_sc as plsc

N = 10000
E = 320000
D = 128
H = 4
NEG_SLOPE = 0.2

NC = 2
NS = 16
N2 = 10240
R = 1024
EPS = E // NS
K = 80
NB = 4
CH = EPS // K
CP = 640


def _sc_edge_body(xp2_hbm, acat_hbm, ei_hbm,
                  num_out, den_out,
                  num_sh, den_sh,
                  sd, dstc, aidx, didx, av1, av2, eev, rows, dd_v,
                  isem, gsem, ssem):
    c = lax.axis_index("c")
    s = lax.axis_index("s")
    r0 = s * CP
    e0 = s * EPS

    def zero_local():
        def zr(i, _):
            for cb in range(D // 16):
                rows[0, i, pl.ds(cb * 16, 16)] = jnp.zeros((16,), jnp.float32)
            return 0
        lax.fori_loop(0, K, zr, 0)

        def ze(j, _):
            dd_v[pl.ds(j * 16, 16)] = jnp.zeros((16,), jnp.float32)
            return 0
        lax.fori_loop(0, CP // 16, ze, 0)

    def idx_descs(ch, j):
        return (
            pltpu.make_async_copy(ei_hbm.at[s * CH + ch], sd.at[j],
                                  isem.at[j]),
        )

    def gather_descs(j):
        return (
            pltpu.make_async_copy(acat_hbm.at[aidx.at[j]], av1.at[j],
                                  gsem.at[j]),
            pltpu.make_async_copy(acat_hbm.at[didx.at[j]], av2.at[j],
                                  gsem.at[j]),
            pltpu.make_async_copy(xp2_hbm.at[aidx.at[j]],
                                  rows.at[j], gsem.at[j]),
        )

    def scat_descs(j):
        return (
            pltpu.make_async_copy(rows.at[j], num_sh.at[dstc.at[j]],
                                  ssem.at[j]),
            pltpu.make_async_copy(eev.at[j], den_sh.at[dstc.at[j]],
                                  ssem.at[j]),
        )

    def compute_idx(h, j):
        def b(t, _):
            sl = pl.ds(t * 16, 16)
            aidx[j, sl] = sd[j, 0, sl] + h * N2
            dv = sd[j, 1, sl]
            dstc[j, sl] = dv
            didx[j, sl] = dv + (H + h) * N2
            return 0
        lax.fori_loop(0, K // 16, b, 0)

    def use(j):
        for d in gather_descs(j):
            d.wait()

        def eb(t, _):
            sl = pl.ds(t * 16, 16)
            e = av1[j, sl] + av2[j, sl]
            e = jnp.where(e >= 0.0, e, e * NEG_SLOPE)
            eev[j, sl] = jnp.exp(e)
            return 0
        lax.fori_loop(0, K // 16, eb, 0)

        def sb(g, _):
            ee16 = eev[j, pl.ds(g * 16, 16)]
            for l in range(16):
                k2 = g * 16 + l
                eek = ee16[l]
                for cb in range(D // 16):
                    sl = pl.ds(cb * 16, 16)
                    rows[j, k2, sl] = rows[j, k2, sl] * eek
            return 0
        lax.fori_loop(0, K // 16, sb, 0)

        for d in scat_descs(j):
            d.start(add=True)

    def accumulate(h):
        def steps(i, jj, guarded):
            def step1(jj=jj):
                for d in scat_descs(jj):
                    d.wait()

            def step2(jj=jj, i=i):
                for d in idx_descs(i + 2, jj):
                    d.start()

            def step3(jj=jj, i=i):
                b = (jj + 3) % 4
                for d in idx_descs(i + 1, b):
                    d.wait()
                compute_idx(h, b)
                for d in gather_descs(b):
                    d.start()

            def step4(jj=jj):
                use((jj + 2) % 4)

            if guarded:
                pl.when(jnp.logical_and(i - 2 >= 0, i - 2 < CH))(step1)
                pl.when(i + 2 < CH)(step2)
                pl.when(jnp.logical_and(i + 1 >= 0, i + 1 < CH))(step3)
                pl.when(jnp.logical_and(i >= 0, i < CH))(step4)
            else:
                step1()
                step2()
                step3()
                step4()

        def mbody(m, _):
            for jj in range(4):
                i = 4 * m + jj - 2
                steady = jnp.logical_and(i - 2 >= 0, i + 2 < CH)

                @pl.when(steady)
                def _(i=i, jj=jj):
                    steps(i, jj, False)

                @pl.when(jnp.logical_not(steady))
                def _(i=i, jj=jj):
                    steps(i, jj, True)
            return 0
        lax.fori_loop(0, (CH + 6) // 4 + 1, mbody, 0)

    def copy_out(h):
        for b in range(CP // K):
            off = r0 + b * K
            pltpu.sync_copy(num_sh.at[pl.ds(off, K), :], rows.at[0])
            pltpu.sync_copy(rows.at[0], num_out.at[h, pl.ds(off, K), :])
        pltpu.sync_copy(den_sh.at[pl.ds(r0, CP)], dd_v)
        pltpu.sync_copy(dd_v, den_out.at[pl.ds(h * N2 + r0, CP)])

    for hp in range(2):
        zero_local()
        for b in range(CP // K):
            off = r0 + b * K
            pltpu.sync_copy(rows.at[0], num_sh.at[pl.ds(off, K), :])
        pltpu.sync_copy(dd_v, den_sh.at[pl.ds(r0, CP)])
        plsc.subcore_barrier()

        accumulate(2 * c + hp)
        plsc.subcore_barrier()

        copy_out(2 * c + hp)
        plsc.subcore_barrier()


def _sc_edge_call(xp2, acat, ei):
    mesh = plsc.VectorSubcoreMesh(core_axis_name="c", subcore_axis_name="s")
    f = pl.kernel(
        _sc_edge_body,
        out_type=[jax.ShapeDtypeStruct((H, N2, D), jnp.float32),
                  jax.ShapeDtypeStruct((H * N2,), jnp.float32)],
        mesh=mesh,
        scratch_types=[
            pltpu.VMEM_SHARED((N2, D), jnp.float32),
            pltpu.VMEM_SHARED((N2,), jnp.float32),
            pltpu.VMEM((NB, 2, K), jnp.int32),
            pltpu.VMEM((NB, K), jnp.int32),
            pltpu.VMEM((NB, K), jnp.int32),
            pltpu.VMEM((NB, K), jnp.int32),
            pltpu.VMEM((NB, K), jnp.float32),
            pltpu.VMEM((NB, K), jnp.float32),
            pltpu.VMEM((NB, K), jnp.float32),
            pltpu.VMEM((NB, K, D), jnp.float32),
            pltpu.VMEM((CP,), jnp.float32),
            pltpu.SemaphoreType.DMA((NB,)),
            pltpu.SemaphoreType.DMA((NB,)),
            pltpu.SemaphoreType.DMA((NB,)),
        ],
        compiler_params=pltpu.CompilerParams(
            needs_layout_passes=False,
            use_tc_tiling_on_sc=False,
        ),
    )
    return f(xp2, acat, ei)


def _tc1_body(x_ref, w_ref, as_ref, ad_ref, xp_ref, a1_ref, a2_ref):
    xb = x_ref[...]
    for h in range(H):
        xp = jnp.dot(xb, w_ref[h], preferred_element_type=jnp.float32)
        xp_ref[h] = xp
        a1_ref[h] = jnp.sum(xp * as_ref[h][None, :], axis=1)
        a2_ref[h] = jnp.sum(xp * ad_ref[h][None, :], axis=1)


def _tc1_call(x, W, att_src, att_dst):
    return pl.pallas_call(
        _tc1_body,
        grid=(N2 // R,),
        in_specs=[
            pl.BlockSpec((R, D), lambda i: (i, 0)),
            pl.BlockSpec((H, D, D), lambda i: (0, 0, 0)),
            pl.BlockSpec((H, D), lambda i: (0, 0)),
            pl.BlockSpec((H, D), lambda i: (0, 0)),
        ],
        out_specs=[
            pl.BlockSpec((H, R, D), lambda i: (0, i, 0)),
            pl.BlockSpec((H, R), lambda i: (0, i)),
            pl.BlockSpec((H, R), lambda i: (0, i)),
        ],
        out_shape=[
            jax.ShapeDtypeStruct((H, N2, D), jnp.float32),
            jax.ShapeDtypeStruct((H, N2), jnp.float32),
            jax.ShapeDtypeStruct((H, N2), jnp.float32),
        ],
    )(x, W, att_src, att_dst)


def _tc2_body(num_ref, den_ref, xp_ref, a1_ref, a2_ref, b_ref, wo_ref,
              bo_ref, o_ref):
    e0 = a1_ref[...] + a2_ref[...]
    e0 = jnp.where(e0 >= 0.0, e0, e0 * NEG_SLOPE)
    ee0 = jnp.exp(e0)
    acc = jnp.zeros((R, D), jnp.float32)
    for h in range(H):
        numh = num_ref[h] + ee0[h][:, None] * xp_ref[h]
        denh = den_ref[h] + ee0[h]
        hh = numh / (denh[:, None] + 1e-16) + b_ref[h][None, :]
        acc = acc + jnp.dot(hh, wo_ref[h * D:(h + 1) * D, :],
                            preferred_element_type=jnp.float32)
    o_ref[...] = jnp.maximum(acc + bo_ref[...], 0.0)


def _tc2_call(num, den, xp, a1, a2, bias, Wo, bo2):
    return pl.pallas_call(
        _tc2_body,
        grid=(N2 // R,),
        in_specs=[
            pl.BlockSpec((H, R, D), lambda i: (0, i, 0)),
            pl.BlockSpec((H, R), lambda i: (0, i)),
            pl.BlockSpec((H, R, D), lambda i: (0, i, 0)),
            pl.BlockSpec((H, R), lambda i: (0, i)),
            pl.BlockSpec((H, R), lambda i: (0, i)),
            pl.BlockSpec((H, D), lambda i: (0, 0)),
            pl.BlockSpec((H * D, D), lambda i: (0, 0)),
            pl.BlockSpec((1, D), lambda i: (0, 0)),
        ],
        out_specs=pl.BlockSpec((R, D), lambda i: (i, 0)),
        out_shape=jax.ShapeDtypeStruct((N, D), jnp.float32),
    )(num, den, xp, a1, a2, bias, Wo, bo2)


def kernel(x, edge_index, W, att_src, att_dst, bias, Wo, bo):
    xp, a1, a2 = _tc1_call(x, W, att_src, att_dst)
    xp2 = xp.reshape(H * N2, D)
    acat = jnp.concatenate([a1.reshape(-1), a2.reshape(-1)])
    ei = jnp.stack([edge_index[0].reshape(NS, CH, K),
                    edge_index[1].reshape(NS, CH, K)],
                   axis=2).reshape(NS * CH, 2, K)
    num, den = _sc_edge_call(xp2, acat, ei)
    return _tc2_call(num, den.reshape(H, N2), xp, a1, a2, bias, Wo,
                     bo.reshape(1, D))


# speedup vs baseline: 1.1064x; 1.1064x over previous
import functools

import jax
import jax.numpy as jnp
from jax import lax
from jax.experimental import pallas as pl
from jax.experimental.pallas import tpu as pltpu
from jax.experimental.pallas import tpu_sc as plsc

N = 10000
E = 320000
D = 128
H = 4
NEG_SLOPE = 0.2

NC = 2
NS = 16
N2 = 10240
R = 1024
EPS = E // NS
K = 80
NB = 4
CH = EPS // K
CP = 640


def _sc_edge_body(xp2_hbm, acat_hbm, ei_hbm,
                  num_out, den_out,
                  num_sh, den_sh,
                  sd, dstc, aidx, didx, av1, av2, eev, rows, dd_v,
                  isem, gsem, ssem):
    c = lax.axis_index("c")
    s = lax.axis_index("s")
    r0 = s * CP
    e0 = s * EPS

    def zero_local():
        def zr(i, _):
            for cb in range(D // 16):
                rows[0, i, pl.ds(cb * 16, 16)] = jnp.zeros((16,), jnp.float32)
            return 0
        lax.fori_loop(0, K, zr, 0)

        def ze(j, _):
            dd_v[pl.ds(j * 16, 16)] = jnp.zeros((16,), jnp.float32)
            return 0
        lax.fori_loop(0, CP // 16, ze, 0)

    def idx_descs(ch, j):
        return (
            pltpu.make_async_copy(ei_hbm.at[s * CH + ch], sd.at[j],
                                  isem.at[j]),
        )

    def gather_descs(j):
        return (
            pltpu.make_async_copy(acat_hbm.at[aidx.at[j]], av1.at[j],
                                  gsem.at[j]),
            pltpu.make_async_copy(acat_hbm.at[didx.at[j]], av2.at[j],
                                  gsem.at[j]),
            pltpu.make_async_copy(xp2_hbm.at[aidx.at[j]],
                                  rows.at[j], gsem.at[j]),
        )

    def scat_descs(j):
        return (
            pltpu.make_async_copy(rows.at[j], num_sh.at[dstc.at[j]],
                                  ssem.at[j]),
            pltpu.make_async_copy(eev.at[j], den_sh.at[dstc.at[j]],
                                  ssem.at[j]),
        )

    def compute_idx(h, j):
        def b(t, _):
            sl = pl.ds(t * 16, 16)
            aidx[j, sl] = sd[j, 0, sl] + h * N2
            dv = sd[j, 1, sl]
            dstc[j, sl] = dv
            didx[j, sl] = dv + (H + h) * N2
            return 0
        lax.fori_loop(0, K // 16, b, 0)

    def use(j):
        for d in gather_descs(j):
            d.wait()

        def eb(t, _):
            sl = pl.ds(t * 16, 16)
            e = av1[j, sl] + av2[j, sl]
            e = jnp.where(e >= 0.0, e, e * NEG_SLOPE)
            eev[j, sl] = jnp.exp(e)
            return 0
        lax.fori_loop(0, K // 16, eb, 0)

        def sb(g, _):
            ee16 = eev[j, pl.ds(g * 16, 16)]
            for l in range(16):
                k2 = g * 16 + l
                eek = ee16[l]
                for cb in range(D // 16):
                    sl = pl.ds(cb * 16, 16)
                    rows[j, k2, sl] = rows[j, k2, sl] * eek
            return 0
        lax.fori_loop(0, K // 16, sb, 0)

        for d in scat_descs(j):
            d.start(add=True)

    def accumulate(h):
        def mbody(m, _):
            for jj in range(4):
                i = 4 * m + jj - 2

                @pl.when(jnp.logical_and(i - 2 >= 0, i - 2 < CH))
                def _(jj=jj):
                    for d in scat_descs(jj):
                        d.wait()

                @pl.when(i + 2 < CH)
                def _(jj=jj, i=i):
                    for d in idx_descs(i + 2, jj):
                        d.start()

                @pl.when(jnp.logical_and(i + 1 >= 0, i + 1 < CH))
                def _(jj=jj, i=i):
                    b = (jj + 3) % 4
                    for d in idx_descs(i + 1, b):
                        d.wait()
                    compute_idx(h, b)
                    for d in gather_descs(b):
                        d.start()

                @pl.when(jnp.logical_and(i >= 0, i < CH))
                def _(jj=jj):
                    use((jj + 2) % 4)
            return 0
        lax.fori_loop(0, (CH + 6) // 4 + 1, mbody, 0)

    def copy_out(h):
        for b in range(CP // K):
            off = r0 + b * K
            pltpu.sync_copy(num_sh.at[pl.ds(off, K), :], rows.at[0])
            pltpu.sync_copy(rows.at[0], num_out.at[h, pl.ds(off, K), :])
        pltpu.sync_copy(den_sh.at[pl.ds(r0, CP)], dd_v)
        pltpu.sync_copy(dd_v, den_out.at[pl.ds(h * N2 + r0, CP)])

    for hp in range(2):
        zero_local()
        for b in range(CP // K):
            off = r0 + b * K
            pltpu.sync_copy(rows.at[0], num_sh.at[pl.ds(off, K), :])
        pltpu.sync_copy(dd_v, den_sh.at[pl.ds(r0, CP)])
        plsc.subcore_barrier()

        accumulate(2 * c + hp)
        plsc.subcore_barrier()

        copy_out(2 * c + hp)
        plsc.subcore_barrier()


def _sc_edge_call(xp2, acat, ei):
    mesh = plsc.VectorSubcoreMesh(core_axis_name="c", subcore_axis_name="s")
    f = pl.kernel(
        _sc_edge_body,
        out_type=[jax.ShapeDtypeStruct((H, N2, D), jnp.float32),
                  jax.ShapeDtypeStruct((H * N2,), jnp.float32)],
        mesh=mesh,
        scratch_types=[
            pltpu.VMEM_SHARED((N2, D), jnp.float32),
            pltpu.VMEM_SHARED((N2,), jnp.float32),
            pltpu.VMEM((NB, 2, K), jnp.int32),
            pltpu.VMEM((NB, K), jnp.int32),
            pltpu.VMEM((NB, K), jnp.int32),
            pltpu.VMEM((NB, K), jnp.int32),
            pltpu.VMEM((NB, K), jnp.float32),
            pltpu.VMEM((NB, K), jnp.float32),
            pltpu.VMEM((NB, K), jnp.float32),
            pltpu.VMEM((NB, K, D), jnp.float32),
            pltpu.VMEM((CP,), jnp.float32),
            pltpu.SemaphoreType.DMA((NB,)),
            pltpu.SemaphoreType.DMA((NB,)),
            pltpu.SemaphoreType.DMA((NB,)),
        ],
        compiler_params=pltpu.CompilerParams(
            needs_layout_passes=False,
            use_tc_tiling_on_sc=False,
        ),
    )
    return f(xp2, acat, ei)


def _tc1_body(x_ref, w_ref, as_ref, ad_ref, xp_ref, a1_ref, a2_ref):
    xb = x_ref[...]
    for h in range(H):
        xp = jnp.dot(xb, w_ref[h], preferred_element_type=jnp.float32)
        xp_ref[h] = xp
        a1_ref[h] = jnp.sum(xp * as_ref[h][None, :], axis=1)
        a2_ref[h] = jnp.sum(xp * ad_ref[h][None, :], axis=1)


def _tc1_call(x, W, att_src, att_dst):
    return pl.pallas_call(
        _tc1_body,
        grid=(N2 // R,),
        in_specs=[
            pl.BlockSpec((R, D), lambda i: (i, 0)),
            pl.BlockSpec((H, D, D), lambda i: (0, 0, 0)),
            pl.BlockSpec((H, D), lambda i: (0, 0)),
            pl.BlockSpec((H, D), lambda i: (0, 0)),
        ],
        out_specs=[
            pl.BlockSpec((H, R, D), lambda i: (0, i, 0)),
            pl.BlockSpec((H, R), lambda i: (0, i)),
            pl.BlockSpec((H, R), lambda i: (0, i)),
        ],
        out_shape=[
            jax.ShapeDtypeStruct((H, N2, D), jnp.float32),
            jax.ShapeDtypeStruct((H, N2), jnp.float32),
            jax.ShapeDtypeStruct((H, N2), jnp.float32),
        ],
    )(x, W, att_src, att_dst)


def _tc2_body(num_ref, den_ref, xp_ref, a1_ref, a2_ref, b_ref, wo_ref,
              bo_ref, o_ref):
    e0 = a1_ref[...] + a2_ref[...]
    e0 = jnp.where(e0 >= 0.0, e0, e0 * NEG_SLOPE)
    ee0 = jnp.exp(e0)
    acc = jnp.zeros((R, D), jnp.float32)
    for h in range(H):
        numh = num_ref[h] + ee0[h][:, None] * xp_ref[h]
        denh = den_ref[h] + ee0[h]
        hh = numh / (denh[:, None] + 1e-16) + b_ref[h][None, :]
        acc = acc + jnp.dot(hh, wo_ref[h * D:(h + 1) * D, :],
                            preferred_element_type=jnp.float32)
    o_ref[...] = jnp.maximum(acc + bo_ref[...], 0.0)


def _tc2_call(num, den, xp, a1, a2, bias, Wo, bo2):
    return pl.pallas_call(
        _tc2_body,
        grid=(N2 // R,),
        in_specs=[
            pl.BlockSpec((H, R, D), lambda i: (0, i, 0)),
            pl.BlockSpec((H, R), lambda i: (0, i)),
            pl.BlockSpec((H, R, D), lambda i: (0, i, 0)),
            pl.BlockSpec((H, R), lambda i: (0, i)),
            pl.BlockSpec((H, R), lambda i: (0, i)),
            pl.BlockSpec((H, D), lambda i: (0, 0)),
            pl.BlockSpec((H * D, D), lambda i: (0, 0)),
            pl.BlockSpec((1, D), lambda i: (0, 0)),
        ],
        out_specs=pl.BlockSpec((R, D), lambda i: (i, 0)),
        out_shape=jax.ShapeDtypeStruct((N, D), jnp.float32),
    )(num, den, xp, a1, a2, bias, Wo, bo2)


def kernel(x, edge_index, W, att_src, att_dst, bias, Wo, bo):
    xp, a1, a2 = _tc1_call(x, W, att_src, att_dst)
    xp2 = xp.reshape(H * N2, D)
    acat = jnp.concatenate([a1.reshape(-1), a2.reshape(-1)])
    ei = jnp.stack([edge_index[0].reshape(NS, CH, K),
                    edge_index[1].reshape(NS, CH, K)],
                   axis=2).reshape(NS * CH, 2, K)
    num, den = _sc_edge_call(xp2, acat, ei)
    return _tc2_call(num, den.reshape(H, N2), xp, a1, a2, bias, Wo,
                     bo.reshape(1, D))


# speedup vs baseline: 1.1627x; 1.0509x over previous
import functools

import jax
import jax.numpy as jnp
from jax import lax
from jax.experimental import pallas as pl
from jax.experimental.pallas import tpu as pltpu
from jax.experimental.pallas import tpu_sc as plsc

N = 10000
E = 320000
D = 128
H = 4
NEG_SLOPE = 0.2

NC = 2
NS = 16
N2 = 10240
R = 1024
EPS = E // NS
K = 80
NB = 4
CH = EPS // K
CP = 640


def _sc_edge_body(xp2_hbm, acat_hbm, ei_hbm,
                  num_out, den_out,
                  num_sh, den_sh,
                  sd, dstc, aidx, didx, av1, av2, eev, rows, dd_v,
                  isem, gsem, ssem):
    c = lax.axis_index("c")
    s = lax.axis_index("s")
    r0 = s * CP
    e0 = s * EPS

    def zero_local():
        def zr(i, _):
            for cb in range(D // 16):
                rows[0, i, pl.ds(cb * 16, 16)] = jnp.zeros((16,), jnp.float32)
            return 0
        lax.fori_loop(0, K, zr, 0)

        def ze(j, _):
            dd_v[pl.ds(j * 16, 16)] = jnp.zeros((16,), jnp.float32)
            return 0
        lax.fori_loop(0, CP // 16, ze, 0)

    def idx_descs(ch, j):
        off = e0 + ch * K
        return (
            pltpu.make_async_copy(ei_hbm.at[0, pl.ds(off, K)], sd.at[j, 0],
                                  isem.at[j]),
            pltpu.make_async_copy(ei_hbm.at[1, pl.ds(off, K)], sd.at[j, 1],
                                  isem.at[j]),
        )

    def gather_descs(j):
        return (
            pltpu.make_async_copy(acat_hbm.at[aidx.at[j]], av1.at[j],
                                  gsem.at[j]),
            pltpu.make_async_copy(acat_hbm.at[didx.at[j]], av2.at[j],
                                  gsem.at[j]),
            pltpu.make_async_copy(xp2_hbm.at[aidx.at[j]],
                                  rows.at[j], gsem.at[j]),
        )

    def scat_descs(j):
        return (
            pltpu.make_async_copy(rows.at[j], num_sh.at[dstc.at[j]],
                                  ssem.at[j]),
            pltpu.make_async_copy(eev.at[j], den_sh.at[dstc.at[j]],
                                  ssem.at[j]),
        )

    def compute_idx(h, j):
        def b(t, _):
            sl = pl.ds(t * 16, 16)
            aidx[j, sl] = sd[j, 0, sl] + h * N2
            dv = sd[j, 1, sl]
            dstc[j, sl] = dv
            didx[j, sl] = dv + (H + h) * N2
            return 0
        lax.fori_loop(0, K // 16, b, 0)

    def use(j):
        for d in gather_descs(j):
            d.wait()

        def eb(t, _):
            sl = pl.ds(t * 16, 16)
            e = av1[j, sl] + av2[j, sl]
            e = jnp.where(e >= 0.0, e, e * NEG_SLOPE)
            eev[j, sl] = jnp.exp(e)
            return 0
        lax.fori_loop(0, K // 16, eb, 0)

        def sb(g, _):
            ee16 = eev[j, pl.ds(g * 16, 16)]
            for l in range(16):
                k2 = g * 16 + l
                eek = ee16[l]
                for cb in range(D // 16):
                    sl = pl.ds(cb * 16, 16)
                    rows[j, k2, sl] = rows[j, k2, sl] * eek
            return 0
        lax.fori_loop(0, K // 16, sb, 0)

        for d in scat_descs(j):
            d.start(add=True)

    def accumulate(h):
        def mbody(m, _):
            for jj in range(4):
                i = 4 * m + jj - 2

                @pl.when(jnp.logical_and(i - 2 >= 0, i - 2 < CH))
                def _(jj=jj):
                    for d in scat_descs(jj):
                        d.wait()

                @pl.when(i + 2 < CH)
                def _(jj=jj, i=i):
                    for d in idx_descs(i + 2, jj):
                        d.start()

                @pl.when(jnp.logical_and(i + 1 >= 0, i + 1 < CH))
                def _(jj=jj, i=i):
                    b = (jj + 3) % 4
                    for d in idx_descs(i + 1, b):
                        d.wait()
                    compute_idx(h, b)
                    for d in gather_descs(b):
                        d.start()

                @pl.when(jnp.logical_and(i >= 0, i < CH))
                def _(jj=jj):
                    use((jj + 2) % 4)
            return 0
        lax.fori_loop(0, (CH + 6) // 4 + 1, mbody, 0)

    def copy_out(h):
        for b in range(CP // K):
            off = r0 + b * K
            pltpu.sync_copy(num_sh.at[pl.ds(off, K), :], rows.at[0])
            pltpu.sync_copy(rows.at[0], num_out.at[h, pl.ds(off, K), :])
        pltpu.sync_copy(den_sh.at[pl.ds(r0, CP)], dd_v)
        pltpu.sync_copy(dd_v, den_out.at[pl.ds(h * N2 + r0, CP)])

    for hp in range(2):
        zero_local()
        for b in range(CP // K):
            off = r0 + b * K
            pltpu.sync_copy(rows.at[0], num_sh.at[pl.ds(off, K), :])
        pltpu.sync_copy(dd_v, den_sh.at[pl.ds(r0, CP)])
        plsc.subcore_barrier()

        accumulate(2 * c + hp)
        plsc.subcore_barrier()

        copy_out(2 * c + hp)
        plsc.subcore_barrier()


def _sc_edge_call(xp2, acat, ei):
    mesh = plsc.VectorSubcoreMesh(core_axis_name="c", subcore_axis_name="s")
    f = pl.kernel(
        _sc_edge_body,
        out_type=[jax.ShapeDtypeStruct((H, N2, D), jnp.float32),
                  jax.ShapeDtypeStruct((H * N2,), jnp.float32)],
        mesh=mesh,
        scratch_types=[
            pltpu.VMEM_SHARED((N2, D), jnp.float32),
            pltpu.VMEM_SHARED((N2,), jnp.float32),
            pltpu.VMEM((NB, 2, K), jnp.int32),
            pltpu.VMEM((NB, K), jnp.int32),
            pltpu.VMEM((NB, K), jnp.int32),
            pltpu.VMEM((NB, K), jnp.int32),
            pltpu.VMEM((NB, K), jnp.float32),
            pltpu.VMEM((NB, K), jnp.float32),
            pltpu.VMEM((NB, K), jnp.float32),
            pltpu.VMEM((NB, K, D), jnp.float32),
            pltpu.VMEM((CP,), jnp.float32),
            pltpu.SemaphoreType.DMA((NB,)),
            pltpu.SemaphoreType.DMA((NB,)),
            pltpu.SemaphoreType.DMA((NB,)),
        ],
        compiler_params=pltpu.CompilerParams(
            needs_layout_passes=False,
            use_tc_tiling_on_sc=False,
        ),
    )
    return f(xp2, acat, ei)


def _tc1_body(x_ref, w_ref, as_ref, ad_ref, xp_ref, ac_ref):
    xb = x_ref[...]
    for h in range(H):
        xp = jnp.dot(xb, w_ref[h], preferred_element_type=jnp.float32)
        xp_ref[h] = xp
        ac_ref[0, h] = jnp.sum(xp * as_ref[h][None, :], axis=1)
        ac_ref[1, h] = jnp.sum(xp * ad_ref[h][None, :], axis=1)


def _tc1_call(x, W, att_src, att_dst):
    return pl.pallas_call(
        _tc1_body,
        grid=(N2 // R,),
        in_specs=[
            pl.BlockSpec((R, D), lambda i: (i, 0)),
            pl.BlockSpec((H, D, D), lambda i: (0, 0, 0)),
            pl.BlockSpec((H, D), lambda i: (0, 0)),
            pl.BlockSpec((H, D), lambda i: (0, 0)),
        ],
        out_specs=[
            pl.BlockSpec((H, R, D), lambda i: (0, i, 0)),
            pl.BlockSpec((2, H, R), lambda i: (0, 0, i)),
        ],
        out_shape=[
            jax.ShapeDtypeStruct((H, N2, D), jnp.float32),
            jax.ShapeDtypeStruct((2, H, N2), jnp.float32),
        ],
    )(x, W, att_src, att_dst)


def _tc2_body(num_ref, den_ref, xp_ref, ac_ref, b_ref, wo_ref,
              bo_ref, o_ref):
    e0 = ac_ref[0] + ac_ref[1]
    e0 = jnp.where(e0 >= 0.0, e0, e0 * NEG_SLOPE)
    ee0 = jnp.exp(e0)
    acc = jnp.zeros((R, D), jnp.float32)
    for h in range(H):
        numh = num_ref[h] + ee0[h][:, None] * xp_ref[h]
        denh = den_ref[h] + ee0[h]
        hh = numh / (denh[:, None] + 1e-16) + b_ref[h][None, :]
        acc = acc + jnp.dot(hh, wo_ref[h * D:(h + 1) * D, :],
                            preferred_element_type=jnp.float32)
    o_ref[...] = jnp.maximum(acc + bo_ref[...], 0.0)


def _tc2_call(num, den, xp, ac, bias, Wo, bo2):
    return pl.pallas_call(
        _tc2_body,
        grid=(N2 // R,),
        in_specs=[
            pl.BlockSpec((H, R, D), lambda i: (0, i, 0)),
            pl.BlockSpec((H, R), lambda i: (0, i)),
            pl.BlockSpec((H, R, D), lambda i: (0, i, 0)),
            pl.BlockSpec((2, H, R), lambda i: (0, 0, i)),
            pl.BlockSpec((H, D), lambda i: (0, 0)),
            pl.BlockSpec((H * D, D), lambda i: (0, 0)),
            pl.BlockSpec((1, D), lambda i: (0, 0)),
        ],
        out_specs=pl.BlockSpec((R, D), lambda i: (i, 0)),
        out_shape=jax.ShapeDtypeStruct((N, D), jnp.float32),
    )(num, den, xp, ac, bias, Wo, bo2)


def kernel(x, edge_index, W, att_src, att_dst, bias, Wo, bo):
    xp, ac = _tc1_call(x, W, att_src, att_dst)
    xp2 = xp.reshape(H * N2, D)
    num, den = _sc_edge_call(xp2, ac.reshape(-1), edge_index)
    return _tc2_call(num, den.reshape(H, N2), xp, ac, bias, Wo,
                     bo.reshape(1, D))
